# Initial kernel scaffold; baseline (speedup 1.0000x reference)
import jax
import jax.numpy as jnp
from jax.experimental import pallas as pl


def kernel(xyz):
    raise NotImplementedError("write your pallas kernel here")


# speedup vs baseline: 13.0879x; 13.0879x over previous
import functools
import numpy as np
import jax
import jax.numpy as jnp
from jax.experimental import pallas as pl
from jax.experimental.pallas import tpu as pltpu

_GS = 32
_CPG = 8
_T = 8
_NROW = 256
_NLANE = 128
_G = 257
_GP = 264
_NGRP = _GP // _CPG


def _centers_np():
    coords = np.arange(-1.0, 1.0 + 1e-6, 0.25, dtype=np.float32)
    gx, gy, gz = np.meshgrid(coords, coords, coords, indexing='ij')
    pts = np.stack([gx.ravel(), gy.ravel(), gz.ravel()], axis=-1)
    keep = np.linalg.norm(pts, axis=-1) <= 1.0 + 1e-6
    return np.asarray(pts[keep], dtype=np.float32)


def _sel_kernel(cref, xref, xbref, oref):
    c = cref[0]
    cx = c[:, 0:1][:, :, None]
    cy = c[:, 1:2][:, :, None]
    cz = c[:, 2:3][:, :, None]
    qq = c[:, 3:4][:, :, None]
    x = xref[0, 0][None]
    y = xref[0, 1][None]
    z = xref[0, 2][None]
    xb = xbref[0, 0][None].astype(jnp.float32)
    yb = xbref[0, 1][None].astype(jnp.float32)
    zb = xbref[0, 2][None].astype(jnp.float32)
    rr = (x * x + y * y) + z * z
    dot = (cx * xb + cy * yb) + cz * zb
    d2 = (qq - 2.0 * dot) + rr

    iota_r = jax.lax.broadcasted_iota(jnp.int32, (_CPG, _NROW, _NLANE), 1)
    big_r = jnp.int32(_NROW)
    w = d2
    vt, rt = [], []
    for _ in range(_T):
        m = jnp.min(w, axis=1)
        eq = w == m[:, None, :]
        r = jnp.min(jnp.where(eq, iota_r, big_r), axis=1)
        vt.append(m)
        rt.append(r)
        w = jnp.where(iota_r == r[:, None, :], jnp.float32(jnp.inf), w)

    lane = jax.lax.broadcasted_iota(jnp.int32, (_CPG, _NLANE), 1)
    gt = [rt[t] * _NLANE + lane for t in range(_T)]
    big_g = jnp.int32(2 ** 30)
    inf = jnp.float32(jnp.inf)

    h = vt[0]
    hg = gt[0]
    ptr = jnp.zeros((_CPG, _NLANE), jnp.int32)
    iota_k = jax.lax.broadcasted_iota(jnp.int32, (_CPG, _GS), 1)
    out = jnp.zeros((_CPG, _GS), jnp.int32)
    for k in range(_GS):
        m = jnp.min(h, axis=1, keepdims=True)
        gsel = jnp.min(jnp.where(h == m, hg, big_g), axis=1,
                       keepdims=True)
        pop = (h == m) & (hg == gsel)
        out = jnp.where(iota_k == k, gsel, out)
        ptr = ptr + pop.astype(jnp.int32)
        newv = jnp.full((_CPG, _NLANE), inf)
        newg = jnp.full((_CPG, _NLANE), big_g)
        for t in range(_T):
            sel = ptr == t
            newv = jnp.where(sel, vt[t], newv)
            newg = jnp.where(sel, gt[t], newg)
        h = jnp.where(pop, newv, h)
        hg = jnp.where(pop, newg, hg)
    oref[0, 0] = out


@functools.partial(jax.jit, static_argnames=("interpret",))
def _select_idx(xt, xtb, cpk, interpret=False):
    return pl.pallas_call(
        _sel_kernel,
        grid=(4, _NGRP),
        in_specs=[
            pl.BlockSpec((1, _CPG, _NLANE), lambda b, g: (g, 0, 0)),
            pl.BlockSpec((1, 3, _NROW, _NLANE), lambda b, g: (b, 0, 0, 0)),
            pl.BlockSpec((1, 3, _NROW, _NLANE), lambda b, g: (b, 0, 0, 0)),
        ],
        out_specs=pl.BlockSpec((1, 1, _CPG, _GS), lambda b, g: (b, g, 0, 0)),
        out_shape=jax.ShapeDtypeStruct((4, _NGRP, _CPG, _GS), jnp.int32),
        interpret=interpret,
    )(cpk, xt, xtb)


def _center_pack():
    cnp = _centers_np()
    cpad = np.full((_GP, 3), 1.0e9, np.float32)
    cpad[:_G] = cnp
    qq = (cpad[:, 0] * cpad[:, 0] + cpad[:, 1] * cpad[:, 1]) + \
        cpad[:, 2] * cpad[:, 2]
    cpk = np.zeros((_NGRP, _CPG, _NLANE), np.float32)
    cpk[:, :, 0] = cpad[:, 0].reshape(_NGRP, _CPG)
    cpk[:, :, 1] = cpad[:, 1].reshape(_NGRP, _CPG)
    cpk[:, :, 2] = cpad[:, 2].reshape(_NGRP, _CPG)
    cpk[:, :, 3] = qq.reshape(_NGRP, _CPG)
    return cnp, cpk


def kernel(xyz, interpret=False):
    B, N, _ = xyz.shape
    cnp, cpk = _center_pack()
    xt = xyz.transpose(0, 2, 1).reshape(B, 3, _NROW, _NLANE)
    xtb = xt.astype(jnp.bfloat16)
    idx = _select_idx(xt, xtb, jnp.asarray(cpk), interpret=interpret)
    idx = idx.reshape(B, _GP * _GS)[:, :_G * _GS]
    center = jnp.broadcast_to(jnp.asarray(cnp)[None], (B, _G, 3))
    idxf = jnp.broadcast_to(idx[:, :, None], (B, _G * _GS, 3))
    nb = jnp.take_along_axis(xyz, idxf, axis=1).reshape(B, _G, _GS, 3)
    nb = nb - center[:, :, None, :]
    return nb, center


# speedup vs baseline: 24.2617x; 1.8538x over previous
import functools
import numpy as np
import jax
import jax.numpy as jnp
from jax import lax
from jax.experimental import pallas as pl
from jax.experimental.pallas import tpu as pltpu
from jax.experimental.pallas import tpu_sc as plsc

_GS = 32
_CPG = 32
_T = 7
_NROW = 256
_NLANE = 128
_G = 257
_GP = 288
_NGRP = _GP // _CPG


def _centers_np():
    coords = np.arange(-1.0, 1.0 + 1e-6, 0.25, dtype=np.float32)
    gx, gy, gz = np.meshgrid(coords, coords, coords, indexing='ij')
    pts = np.stack([gx.ravel(), gy.ravel(), gz.ravel()], axis=-1)
    keep = np.linalg.norm(pts, axis=-1) <= 1.0 + 1e-6
    return np.asarray(pts[keep], dtype=np.float32)


def _sel_kernel(cref, rrref, xbref, oref):
    c = cref[0]
    cx = c[:, 0:1][:, :, None]
    cy = c[:, 1:2][:, :, None]
    cz = c[:, 2:3][:, :, None]
    qq = c[:, 3:4][:, :, None]
    rr = rrref[0][None]
    xb = xbref[0, 0][None].astype(jnp.float32)
    yb = xbref[0, 1][None].astype(jnp.float32)
    zb = xbref[0, 2][None].astype(jnp.float32)
    dot = (cx * xb + cy * yb) + cz * zb
    d2 = (qq - 2.0 * dot) + rr

    iota_r = jax.lax.broadcasted_iota(jnp.int32, (_CPG, _NROW, _NLANE), 1)
    big_r = jnp.int32(_NROW)
    w = d2
    vt, rt = [], []
    for _ in range(_T):
        m = jnp.min(w, axis=1)
        eq = w == m[:, None, :]
        r = jnp.min(jnp.where(eq, iota_r, big_r), axis=1)
        vt.append(m)
        rt.append(r)
        w = jnp.where(iota_r == r[:, None, :], jnp.float32(jnp.inf), w)

    lane = jax.lax.broadcasted_iota(jnp.int32, (_CPG, _NLANE), 1)
    gt = [rt[t] * _NLANE + lane for t in range(_T)]
    big_g = jnp.int32(2 ** 30)
    inf = jnp.float32(jnp.inf)

    h = vt[0]
    hg = gt[0]
    ptr = jnp.zeros((_CPG, _NLANE), jnp.int32)
    iota_k = jax.lax.broadcasted_iota(jnp.int32, (_CPG, _GS), 1)
    out = jnp.zeros((_CPG, _GS), jnp.int32)
    for k in range(_GS):
        m = jnp.min(h, axis=1, keepdims=True)
        gsel = jnp.min(jnp.where(h == m, hg, big_g), axis=1,
                       keepdims=True)
        pop = (h == m) & (hg == gsel)
        out = jnp.where(iota_k == k, gsel, out)
        ptr = ptr + pop.astype(jnp.int32)
        newv = jnp.full((_CPG, _NLANE), inf)
        newg = jnp.full((_CPG, _NLANE), big_g)
        for t in range(_T):
            sel = ptr == t
            newv = jnp.where(sel, vt[t], newv)
            newg = jnp.where(sel, gt[t], newg)
        h = jnp.where(pop, newv, h)
        hg = jnp.where(pop, newg, hg)
    oref[0, 0] = out + pl.program_id(0) * 32768


@functools.partial(jax.jit, static_argnames=("interpret",))
def _select_idx(rr4, xtb, cpk, interpret=False):
    return pl.pallas_call(
        _sel_kernel,
        grid=(4, _NGRP),
        in_specs=[
            pl.BlockSpec((1, _CPG, _NLANE), lambda b, g: (g, 0, 0)),
            pl.BlockSpec((1, _NROW, _NLANE), lambda b, g: (b, 0, 0)),
            pl.BlockSpec((1, 3, _NROW, _NLANE), lambda b, g: (b, 0, 0, 0)),
        ],
        out_specs=pl.BlockSpec((1, 1, _CPG, _GS), lambda b, g: (b, g, 0, 0)),
        out_shape=jax.ShapeDtypeStruct((4, _NGRP, _CPG, _GS), jnp.int32),
        interpret=interpret,
    )(cpk, rr4, xtb)


_NROWS_OUT = 4 * _GP * _GS
_DPAD = 16
_NW = 32
_RPW = _NROWS_OUT // _NW
_CHUNK = 64
_NCHUNK = _RPW // _CHUNK
assert _CHUNK * _NCHUNK == _RPW and _CHUNK % 8 == 0


def _gather_body(tab, idxh, cen, out, idx_v, rows_v, cent_v, sem):
    info = plsc.get_sparse_core_info()
    wid = lax.axis_index("s") * info.num_cores + lax.axis_index("c")
    base = wid * _RPW
    pltpu.sync_copy(idxh.at[pl.ds(base, _RPW)], idx_v)
    pltpu.sync_copy(cen.at[pl.ds(base, _RPW)], cent_v)
    cps = [
        pltpu.async_copy(
            tab.at[idx_v.at[pl.ds(j * _CHUNK, _CHUNK)]],
            rows_v.at[pl.ds(j * _CHUNK, _CHUNK)], sem)
        for j in range(_NCHUNK)
    ]
    for cp in cps:
        cp.wait()

    def body(i, carry):
        rows_v[i, :] = rows_v[i, :] - cent_v[i, :]
        return carry

    lax.fori_loop(0, _RPW, body, 0)
    pltpu.sync_copy(rows_v, out.at[pl.ds(base, _RPW)])


@jax.jit
def _gather_sub(tab, idxf, cexp):
    mesh = plsc.VectorSubcoreMesh(core_axis_name="c", subcore_axis_name="s")
    return pl.kernel(
        _gather_body,
        mesh=mesh,
        compiler_params=pltpu.CompilerParams(use_tc_tiling_on_sc=False),
        out_type=jax.ShapeDtypeStruct((_NROWS_OUT, _DPAD), jnp.float32),
        scratch_types=[
            pltpu.VMEM((_RPW,), jnp.int32),
            pltpu.VMEM((_RPW, _DPAD), jnp.float32),
            pltpu.VMEM((_RPW, _DPAD), jnp.float32),
            pltpu.SemaphoreType.DMA,
        ],
    )(tab, idxf, cexp)


def _center_pack():
    cnp = _centers_np()
    cpad = np.full((_GP, 3), 1.0e9, np.float32)
    cpad[:_G] = cnp
    qq = (cpad[:, 0] * cpad[:, 0] + cpad[:, 1] * cpad[:, 1]) + \
        cpad[:, 2] * cpad[:, 2]
    cpk = np.zeros((_NGRP, _CPG, _NLANE), np.float32)
    cpk[:, :, 0] = cpad[:, 0].reshape(_NGRP, _CPG)
    cpk[:, :, 1] = cpad[:, 1].reshape(_NGRP, _CPG)
    cpk[:, :, 2] = cpad[:, 2].reshape(_NGRP, _CPG)
    cpk[:, :, 3] = qq.reshape(_NGRP, _CPG)
    return cnp, cpk


@functools.lru_cache(maxsize=1)
def _center_exp():
    cnp, cpk = _center_pack()
    cexp = np.zeros((4, _GP, _GS, _DPAD), np.float32)
    cexp[:, :_G, :, :3] = cnp[None, :, None, :]
    return cnp, cpk, cexp.reshape(_NROWS_OUT, _DPAD)


def kernel(xyz, interpret=False):
    B, N, _ = xyz.shape
    cnp, cpk, cexp = _center_exp()
    rr4 = ((xyz[:, :, 0] * xyz[:, :, 0] + xyz[:, :, 1] * xyz[:, :, 1])
           + xyz[:, :, 2] * xyz[:, :, 2]).reshape(B, _NROW, _NLANE)
    xtb = xyz.astype(jnp.bfloat16).transpose(0, 2, 1) \
        .reshape(B, 3, _NROW, _NLANE)
    idx = _select_idx(rr4, xtb, jnp.asarray(cpk), interpret=interpret)
    tab = jnp.pad(xyz, ((0, 0), (0, 0), (0, _DPAD - 3))) \
        .reshape(B * N, _DPAD)
    out = _gather_sub(tab, idx.reshape(_NROWS_OUT), jnp.asarray(cexp))
    nb = out.reshape(B, _GP, _GS, _DPAD)[:, :_G, :, :3]
    center = jnp.broadcast_to(jnp.asarray(cnp)[None], (B, _G, 3))
    return nb, center


# speedup vs baseline: 25.0530x; 1.0326x over previous
import functools
import numpy as np
import jax
import jax.numpy as jnp
from jax import lax
from jax.experimental import pallas as pl
from jax.experimental.pallas import tpu as pltpu
from jax.experimental.pallas import tpu_sc as plsc

_GS = 32
_CPG = 48
_T = 7
_NROW = 256
_NLANE = 128
_G = 257
_GP = 288
_NGRP = _GP // _CPG


def _centers_np():
    coords = np.arange(-1.0, 1.0 + 1e-6, 0.25, dtype=np.float32)
    gx, gy, gz = np.meshgrid(coords, coords, coords, indexing='ij')
    pts = np.stack([gx.ravel(), gy.ravel(), gz.ravel()], axis=-1)
    keep = np.linalg.norm(pts, axis=-1) <= 1.0 + 1e-6
    return np.asarray(pts[keep], dtype=np.float32)


def _sel_kernel(cref, rrref, xbref, oref):
    c = cref[0]
    cx = c[:, 0:1][:, :, None]
    cy = c[:, 1:2][:, :, None]
    cz = c[:, 2:3][:, :, None]
    qq = c[:, 3:4][:, :, None]
    rr = rrref[0][None]
    xb = xbref[0, 0][None].astype(jnp.float32)
    yb = xbref[0, 1][None].astype(jnp.float32)
    zb = xbref[0, 2][None].astype(jnp.float32)
    dot = (cx * xb + cy * yb) + cz * zb
    d2 = (qq - 2.0 * dot) + rr

    iota_r = jax.lax.broadcasted_iota(jnp.int32, (_CPG, _NROW, _NLANE), 1)
    big_r = jnp.int32(_NROW)
    w = d2
    vt, rt = [], []
    for _ in range(_T):
        m = jnp.min(w, axis=1)
        eq = w == m[:, None, :]
        r = jnp.min(jnp.where(eq, iota_r, big_r), axis=1)
        vt.append(m)
        rt.append(r)
        w = jnp.where(iota_r == r[:, None, :], jnp.float32(jnp.inf), w)

    lane = jax.lax.broadcasted_iota(jnp.int32, (_CPG, _NLANE), 1)
    gt = [rt[t] * _NLANE + lane for t in range(_T)]
    big_g = jnp.int32(2 ** 30)
    inf = jnp.float32(jnp.inf)

    h = vt[0]
    hg = gt[0]
    ptr = jnp.zeros((_CPG, _NLANE), jnp.int32)
    iota_k = jax.lax.broadcasted_iota(jnp.int32, (_CPG, _GS), 1)
    out = jnp.zeros((_CPG, _GS), jnp.int32)
    for k in range(_GS):
        m = jnp.min(h, axis=1, keepdims=True)
        gsel = jnp.min(jnp.where(h == m, hg, big_g), axis=1,
                       keepdims=True)
        pop = (h == m) & (hg == gsel)
        out = jnp.where(iota_k == k, gsel, out)
        ptr = ptr + pop.astype(jnp.int32)
        newv = jnp.full((_CPG, _NLANE), inf)
        newg = jnp.full((_CPG, _NLANE), big_g)
        for t in range(_T):
            sel = ptr == t
            newv = jnp.where(sel, vt[t], newv)
            newg = jnp.where(sel, gt[t], newg)
        h = jnp.where(pop, newv, h)
        hg = jnp.where(pop, newg, hg)
    oref[0, 0] = out + pl.program_id(0) * 32768


@functools.partial(jax.jit, static_argnames=("interpret",))
def _select_idx(rr4, xtb, cpk, interpret=False):
    return pl.pallas_call(
        _sel_kernel,
        grid=(4, _NGRP),
        in_specs=[
            pl.BlockSpec((1, _CPG, _NLANE), lambda b, g: (g, 0, 0)),
            pl.BlockSpec((1, _NROW, _NLANE), lambda b, g: (b, 0, 0)),
            pl.BlockSpec((1, 3, _NROW, _NLANE), lambda b, g: (b, 0, 0, 0)),
        ],
        out_specs=pl.BlockSpec((1, 1, _CPG, _GS), lambda b, g: (b, g, 0, 0)),
        out_shape=jax.ShapeDtypeStruct((4, _NGRP, _CPG, _GS), jnp.int32),
        interpret=interpret,
    )(cpk, rr4, xtb)


_NROWS_OUT = 4 * _GP * _GS
_DPAD = 16
_NW = 32
_RPW = _NROWS_OUT // _NW
_CHUNK = 64
_NCHUNK = _RPW // _CHUNK
assert _CHUNK * _NCHUNK == _RPW and _CHUNK % 8 == 0


def _gather_body(tab, idxh, cen, out, idx_v, rows_v, cent_v, sem):
    info = plsc.get_sparse_core_info()
    wid = lax.axis_index("s") * info.num_cores + lax.axis_index("c")
    base = wid * _RPW
    pltpu.sync_copy(idxh.at[pl.ds(base, _RPW)], idx_v)
    pltpu.sync_copy(cen.at[pl.ds(base, _RPW)], cent_v)
    cps = [
        pltpu.async_copy(
            tab.at[idx_v.at[pl.ds(j * _CHUNK, _CHUNK)]],
            rows_v.at[pl.ds(j * _CHUNK, _CHUNK)], sem)
        for j in range(_NCHUNK)
    ]
    for cp in cps:
        cp.wait()

    def body(i, carry):
        rows_v[i, :] = rows_v[i, :] - cent_v[i, :]
        return carry

    lax.fori_loop(0, _RPW, body, 0)
    pltpu.sync_copy(rows_v, out.at[pl.ds(base, _RPW)])


@jax.jit
def _gather_sub(tab, idxf, cexp):
    mesh = plsc.VectorSubcoreMesh(core_axis_name="c", subcore_axis_name="s")
    return pl.kernel(
        _gather_body,
        mesh=mesh,
        compiler_params=pltpu.CompilerParams(use_tc_tiling_on_sc=False),
        out_type=jax.ShapeDtypeStruct((_NROWS_OUT, _DPAD), jnp.float32),
        scratch_types=[
            pltpu.VMEM((_RPW,), jnp.int32),
            pltpu.VMEM((_RPW, _DPAD), jnp.float32),
            pltpu.VMEM((_RPW, _DPAD), jnp.float32),
            pltpu.SemaphoreType.DMA,
        ],
    )(tab, idxf, cexp)


def _center_pack():
    cnp = _centers_np()
    cpad = np.full((_GP, 3), 1.0e9, np.float32)
    cpad[:_G] = cnp
    qq = (cpad[:, 0] * cpad[:, 0] + cpad[:, 1] * cpad[:, 1]) + \
        cpad[:, 2] * cpad[:, 2]
    cpk = np.zeros((_NGRP, _CPG, _NLANE), np.float32)
    cpk[:, :, 0] = cpad[:, 0].reshape(_NGRP, _CPG)
    cpk[:, :, 1] = cpad[:, 1].reshape(_NGRP, _CPG)
    cpk[:, :, 2] = cpad[:, 2].reshape(_NGRP, _CPG)
    cpk[:, :, 3] = qq.reshape(_NGRP, _CPG)
    return cnp, cpk


@functools.lru_cache(maxsize=1)
def _center_exp():
    cnp, cpk = _center_pack()
    cexp = np.zeros((4, _GP, _GS, _DPAD), np.float32)
    cexp[:, :_G, :, :3] = cnp[None, :, None, :]
    return cnp, cpk, cexp.reshape(_NROWS_OUT, _DPAD)


def kernel(xyz, interpret=False):
    B, N, _ = xyz.shape
    cnp, cpk, cexp = _center_exp()
    rr4 = ((xyz[:, :, 0] * xyz[:, :, 0] + xyz[:, :, 1] * xyz[:, :, 1])
           + xyz[:, :, 2] * xyz[:, :, 2]).reshape(B, _NROW, _NLANE)
    xtb = xyz.astype(jnp.bfloat16).transpose(0, 2, 1) \
        .reshape(B, 3, _NROW, _NLANE)
    idx = _select_idx(rr4, xtb, jnp.asarray(cpk), interpret=interpret)
    tab = jnp.pad(xyz, ((0, 0), (0, 0), (0, _DPAD - 3))) \
        .reshape(B * N, _DPAD)
    out = _gather_sub(tab, idx.reshape(_NROWS_OUT), jnp.asarray(cexp))
    nb = out.reshape(B, _GP, _GS, _DPAD)[:, :_G, :, :3]
    center = jnp.broadcast_to(jnp.asarray(cnp)[None], (B, _G, 3))
    return nb, center
